# baseline (speedup 1.0000x reference)
import functools

import jax
import jax.numpy as jnp
from jax import lax
from jax.experimental import pallas as pl
from jax.experimental.pallas import tpu as pltpu
from jax.experimental.pallas import tpu_sc as plsc


def _make_sc_gather(V, E, N):
    info = plsc.get_sparse_core_info()
    NW = info.num_cores * info.num_subcores
    NC = info.num_cores
    per_w = N // NW
    CH = 80
    n_ch = per_w // CH
    assert per_w * NW == N and n_ch * CH == per_w

    mesh = plsc.VectorSubcoreMesh(core_axis_name="c", subcore_axis_name="s")

    @functools.partial(
        pl.kernel,
        mesh=mesh,
        out_type=jax.ShapeDtypeStruct((N, E), jnp.float32),
        scratch_types=[
            pltpu.VMEM((n_ch, CH), jnp.int32),
            pltpu.VMEM((per_w, E), jnp.float32),
            pltpu.SemaphoreType.DMA,
        ],
        compiler_params=pltpu.CompilerParams(use_tc_tiling_on_sc=False),
    )
    def gather_k(table_hbm, idx_hbm, out_hbm, idx_v, rows_v, sem):
        wid = lax.axis_index("s") * NC + lax.axis_index("c")
        base = wid * per_w
        pltpu.sync_copy(idx_hbm.at[wid], idx_v)
        copies = []
        for j in range(n_ch):
            copies.append(
                pltpu.async_copy(
                    table_hbm.at[idx_v.at[j]],
                    rows_v.at[pl.ds(j * CH, CH)],
                    sem,
                )
            )
        for c in copies:
            c.wait()
        pltpu.sync_copy(rows_v, out_hbm.at[pl.ds(base, per_w)])

    return gather_k


def _gru_body(L, H, lens_ref, wih_ref, whh_ref, bih_ref, bhh_ref, x_ref,
              out_ref, hid_ref, h_scr):
    t = pl.program_id(0)

    @pl.when(t == 0)
    def _init():
        h_scr[...] = jnp.zeros_like(h_scr)

    h = h_scr[...]
    x_t = x_ref[0]
    gi = jnp.dot(x_t, wih_ref[...], preferred_element_type=jnp.float32)
    gi = gi + bih_ref[...]
    gh = jnp.dot(h, whh_ref[...], preferred_element_type=jnp.float32)
    gh = gh + bhh_ref[...]
    r = jax.nn.sigmoid(gi[:, :H] + gh[:, :H])
    z = jax.nn.sigmoid(gi[:, H:2 * H] + gh[:, H:2 * H])
    n = jnp.tanh(gi[:, 2 * H:] + r * gh[:, 2 * H:])
    h_new = (1.0 - z) * n + z * h
    valid = t < lens_ref[...]
    h_keep = jnp.where(valid, h_new, h)
    h_scr[...] = h_keep
    out_ref[:, pl.ds(t % 8, 1), :] = jnp.where(valid, h_new, 0.0)[:, None, :]

    @pl.when(t == L - 1)
    def _fin():
        hid_ref[...] = h_keep


def _make_gru(B, L, E, H):
    return pl.pallas_call(
        functools.partial(_gru_body, L, H),
        grid=(L,),
        in_specs=[
            pl.BlockSpec((B, 1), lambda t: (0, 0)),
            pl.BlockSpec((E, 3 * H), lambda t: (0, 0)),
            pl.BlockSpec((H, 3 * H), lambda t: (0, 0)),
            pl.BlockSpec((1, 3 * H), lambda t: (0, 0)),
            pl.BlockSpec((1, 3 * H), lambda t: (0, 0)),
            pl.BlockSpec((1, B, E), lambda t: (t, 0, 0)),
        ],
        out_specs=[
            pl.BlockSpec((B, 8, H), lambda t: (0, t // 8, 0)),
            pl.BlockSpec((B, H), lambda t: (0, 0)),
        ],
        out_shape=[
            jax.ShapeDtypeStruct((B, L, H), jnp.float32),
            jax.ShapeDtypeStruct((B, H), jnp.float32),
        ],
        scratch_shapes=[pltpu.VMEM((B, H), jnp.float32)],
    )


def kernel(input_seqs, seq_lengths, table, W_ih, W_hh, b_ih, b_hh):
    B, L = input_seqs.shape
    V, E = table.shape
    H = W_hh.shape[1]

    order = jnp.argsort(-seq_lengths)
    lengths = seq_lengths[order]
    seqs = input_seqs[order]

    info = plsc.get_sparse_core_info()
    NW = info.num_cores * info.num_subcores
    N = B * L
    per_w = N // NW
    CH = 80
    idx = jnp.transpose(seqs, (1, 0)).reshape(NW, per_w // CH, CH)

    emb = _make_sc_gather(V, E, N)(table, idx)
    x = emb.reshape(L, B, E)

    outputs, hT = _make_gru(B, L, E, H)(
        lengths[:, None],
        jnp.transpose(W_ih, (1, 0)),
        jnp.transpose(W_hh, (1, 0)),
        b_ih[None, :],
        b_hh[None, :],
        x,
    )
    return outputs, hT[None, :, :]


# speedup vs baseline: 1.3945x; 1.3945x over previous
import functools

import jax
import jax.numpy as jnp
from jax import lax
from jax.experimental import pallas as pl
from jax.experimental.pallas import tpu as pltpu
from jax.experimental.pallas import tpu_sc as plsc


def _make_sc_gather(V, E, N):
    info = plsc.get_sparse_core_info()
    NW = info.num_cores * info.num_subcores
    NC = info.num_cores
    per_w = N // NW
    CH = 80
    n_ch = per_w // CH
    assert per_w * NW == N and n_ch * CH == per_w

    mesh = plsc.VectorSubcoreMesh(core_axis_name="c", subcore_axis_name="s")

    @functools.partial(
        pl.kernel,
        mesh=mesh,
        out_type=jax.ShapeDtypeStruct((N, E), jnp.float32),
        scratch_types=[
            pltpu.VMEM((n_ch, CH), jnp.int32),
            pltpu.VMEM((per_w, E), jnp.float32),
            pltpu.SemaphoreType.DMA,
        ],
        compiler_params=pltpu.CompilerParams(use_tc_tiling_on_sc=False),
    )
    def gather_k(table_hbm, idx_hbm, out_hbm, idx_v, rows_v, sem):
        wid = lax.axis_index("s") * NC + lax.axis_index("c")
        base = wid * per_w
        pltpu.sync_copy(idx_hbm.at[wid], idx_v)
        copies = []
        for j in range(n_ch):
            copies.append(
                pltpu.async_copy(
                    table_hbm.at[idx_v.at[j]],
                    rows_v.at[pl.ds(j * CH, CH)],
                    sem,
                )
            )
        for c in copies:
            c.wait()
        pltpu.sync_copy(rows_v, out_hbm.at[pl.ds(base, per_w)])

    return gather_k


def _gru_body(L, H, lens_ref, wih_ref, whh_ref, bih_ref, bhh_ref, x_ref,
              out_ref, hid_ref, h_scr):
    t = pl.program_id(0)

    @pl.when(t == 0)
    def _init():
        h_scr[...] = jnp.zeros_like(h_scr)

    h = h_scr[...]
    x_t = x_ref[0]
    gi = jnp.dot(x_t, wih_ref[...], preferred_element_type=jnp.float32)
    gi = gi + bih_ref[...]
    gh = jnp.dot(h, whh_ref[...], preferred_element_type=jnp.float32)
    gh = gh + bhh_ref[...]
    P = 2 * H
    r = jax.nn.sigmoid(gi[:, :P] + gh[:, :P])
    z = jax.nn.sigmoid(gi[:, P:2 * P] + gh[:, P:2 * P])
    n = jnp.tanh(gi[:, 2 * P:] + r * gh[:, 2 * P:])
    h_new = (1.0 - z) * n + z * h
    valid = t < lens_ref[...]
    h_keep = jnp.where(valid, h_new, h)
    h_scr[...] = h_keep
    out_ref[0] = jnp.where(valid, h_new, 0.0)

    @pl.when(t == L - 1)
    def _fin():
        hid_ref[...] = h_keep


def _make_gru(B, L, E, H):
    B2, P = B // 2, 2 * H
    return pl.pallas_call(
        functools.partial(_gru_body, L, H),
        grid=(L,),
        in_specs=[
            pl.BlockSpec((B2, P), lambda t: (0, 0)),
            pl.BlockSpec((2 * E, 3 * P), lambda t: (0, 0)),
            pl.BlockSpec((P, 3 * P), lambda t: (0, 0)),
            pl.BlockSpec((1, 3 * P), lambda t: (0, 0)),
            pl.BlockSpec((1, 3 * P), lambda t: (0, 0)),
            pl.BlockSpec((1, B2, 2 * E), lambda t: (t, 0, 0)),
        ],
        out_specs=[
            pl.BlockSpec((1, B2, P), lambda t: (t, 0, 0)),
            pl.BlockSpec((B2, P), lambda t: (0, 0)),
        ],
        out_shape=[
            jax.ShapeDtypeStruct((L, B2, P), jnp.float32),
            jax.ShapeDtypeStruct((B2, P), jnp.float32),
        ],
        scratch_shapes=[pltpu.VMEM((B2, P), jnp.float32)],
    )


def _pack_weights(Wt, E, H):
    Z = jnp.zeros((E, H), Wt.dtype)
    top = jnp.concatenate(
        [Wt[:, :H], Z, Wt[:, H:2 * H], Z, Wt[:, 2 * H:], Z], axis=1)
    bot = jnp.concatenate(
        [Z, Wt[:, :H], Z, Wt[:, H:2 * H], Z, Wt[:, 2 * H:]], axis=1)
    return jnp.concatenate([top, bot], axis=0)


def _pack_bias(b, H):
    return jnp.concatenate(
        [b[:H], b[:H], b[H:2 * H], b[H:2 * H], b[2 * H:], b[2 * H:]])[None]


def kernel(input_seqs, seq_lengths, table, W_ih, W_hh, b_ih, b_hh):
    B, L = input_seqs.shape
    V, E = table.shape
    H = W_hh.shape[1]

    order = jnp.argsort(-seq_lengths)
    lengths = seq_lengths[order]
    seqs = input_seqs[order]

    info = plsc.get_sparse_core_info()
    NW = info.num_cores * info.num_subcores
    N = B * L
    per_w = N // NW
    CH = 80
    idx = jnp.transpose(seqs, (1, 0)).reshape(NW, per_w // CH, CH)

    emb = _make_sc_gather(V, E, N)(table, idx)
    x = emb.reshape(L, B // 2, 2 * E)

    lens2 = jnp.repeat(lengths, H).reshape(B // 2, 2 * H)
    out_p, hid_p = _make_gru(B, L, E, H)(
        lens2,
        _pack_weights(jnp.transpose(W_ih, (1, 0)), E, H),
        _pack_weights(jnp.transpose(W_hh, (1, 0)), H, H),
        _pack_bias(b_ih, H),
        _pack_bias(b_hh, H),
        x,
    )
    outputs = jnp.transpose(out_p.reshape(L, B, H), (1, 0, 2))
    hidden = hid_p.reshape(B, H)[None, :, :]
    return outputs, hidden


# speedup vs baseline: 1.4713x; 1.0550x over previous
import functools

import jax
import jax.numpy as jnp
from jax import lax
from jax.experimental import pallas as pl
from jax.experimental.pallas import tpu as pltpu
from jax.experimental.pallas import tpu_sc as plsc


def _fmt_body(x_ref, o_ref):
    xt = jnp.transpose(x_ref[...], (1, 0))
    half = xt.shape[0] // 2
    o_ref[...] = jnp.concatenate([xt[:half], xt[half:]], axis=1)


def _make_table_fmt(V_pad, E):
    CH = 1024
    G = V_pad // CH
    return pl.pallas_call(
        _fmt_body,
        grid=(G,),
        in_specs=[pl.BlockSpec((E, CH), lambda i: (0, i))],
        out_specs=pl.BlockSpec((CH // 2, 2 * E), lambda i: (i, 0)),
        out_shape=jax.ShapeDtypeStruct((V_pad // 2, 2 * E), jnp.float32),
    )


def _make_sc_gather(V, E, N):
    info = plsc.get_sparse_core_info()
    NW = info.num_cores * info.num_subcores
    NC = info.num_cores
    per_w = N // NW
    CH = 80
    n_ch = per_w // CH
    assert per_w * NW == N and n_ch * CH == per_w

    mesh = plsc.VectorSubcoreMesh(core_axis_name="c", subcore_axis_name="s")

    @functools.partial(
        pl.kernel,
        mesh=mesh,
        out_type=jax.ShapeDtypeStruct((N, E), jnp.float32),
        scratch_types=[
            pltpu.VMEM((n_ch, CH), jnp.int32),
            pltpu.VMEM((per_w, E), jnp.float32),
            pltpu.SemaphoreType.DMA,
        ],
        compiler_params=pltpu.CompilerParams(use_tc_tiling_on_sc=False),
    )
    def gather_k(table_hbm, idx_hbm, out_hbm, idx_v, rows_v, sem):
        wid = lax.axis_index("s") * NC + lax.axis_index("c")
        base = wid * per_w
        pltpu.sync_copy(idx_hbm.at[wid], idx_v)
        copies = []
        for j in range(n_ch):
            copies.append(
                pltpu.async_copy(
                    table_hbm.at[idx_v.at[j]],
                    rows_v.at[pl.ds(j * CH, CH)],
                    sem,
                )
            )
        for c in copies:
            c.wait()
        pltpu.sync_copy(rows_v, out_hbm.at[pl.ds(base, per_w)])

    return gather_k


def _gru_body(B, E, L, H, lens_ref, wih_ref, whh_ref, bih_ref, bhh_ref,
              x_ref, out_ref, hid_ref, h_scr):
    t = pl.program_id(0)

    @pl.when(t == 0)
    def _init():
        h_scr[...] = jnp.zeros_like(h_scr)

    h = h_scr[...]
    v = x_ref[0]
    xt = jnp.concatenate([v[:, :E], v[:, E:]], axis=0)
    gi = lax.dot_general(wih_ref[...], xt, (((1,), (1,)), ((), ())),
                         preferred_element_type=jnp.float32)
    gi = gi + bih_ref[...]
    gh = jnp.dot(whh_ref[...], h, preferred_element_type=jnp.float32)
    gh = gh + bhh_ref[...]
    r = jax.nn.sigmoid(gi[:H] + gh[:H])
    z = jax.nn.sigmoid(gi[H:2 * H] + gh[H:2 * H])
    n = jnp.tanh(gi[2 * H:] + r * gh[2 * H:])
    h_new = (1.0 - z) * n + z * h
    valid = t < lens_ref[...]
    h_keep = jnp.where(valid, h_new, h)
    h_scr[...] = h_keep
    out_ref[0] = jnp.where(valid, h_new, 0.0)

    @pl.when(t == L - 1)
    def _fin():
        hid_ref[...] = h_keep


def _make_gru(B, L, E, H):
    return pl.pallas_call(
        functools.partial(_gru_body, B, E, L, H),
        grid=(L,),
        in_specs=[
            pl.BlockSpec((1, B), lambda t: (0, 0)),
            pl.BlockSpec((3 * H, E), lambda t: (0, 0)),
            pl.BlockSpec((3 * H, H), lambda t: (0, 0)),
            pl.BlockSpec((3 * H, 1), lambda t: (0, 0)),
            pl.BlockSpec((3 * H, 1), lambda t: (0, 0)),
            pl.BlockSpec((1, B // 2, 2 * E), lambda t: (t, 0, 0)),
        ],
        out_specs=[
            pl.BlockSpec((1, H, B), lambda t: (t, 0, 0)),
            pl.BlockSpec((H, B), lambda t: (0, 0)),
        ],
        out_shape=[
            jax.ShapeDtypeStruct((L, H, B), jnp.float32),
            jax.ShapeDtypeStruct((H, B), jnp.float32),
        ],
        scratch_shapes=[pltpu.VMEM((H, B), jnp.float32)],
    )


def kernel(input_seqs, seq_lengths, table, W_ih, W_hh, b_ih, b_hh):
    B, L = input_seqs.shape
    V, E = table.shape
    H = W_hh.shape[1]

    order = jnp.argsort(-seq_lengths)
    lengths = seq_lengths[order]
    seqs = input_seqs[order]

    info = plsc.get_sparse_core_info()
    NW = info.num_cores * info.num_subcores
    N = B * L
    per_w = N // NW
    CH = 80
    V_pad = -(-V // 1024) * 1024
    v = jnp.transpose(seqs, (1, 0))
    p = v % 1024
    sig = (v - p) + 2 * (p % 512) + p // 512
    binv = 512 * (jnp.arange(B) % 2) + jnp.arange(B) // 2
    sig = sig[:, binv]
    idx = sig.reshape(NW, per_w // CH, CH)

    table_fmt = _make_table_fmt(V_pad, E)(jnp.transpose(table, (1, 0)))
    table_rm = table_fmt.reshape(V_pad, E)

    emb = _make_sc_gather(V_pad, E, N)(table_rm, idx)
    x = emb.reshape(L, B // 2, 2 * E)

    out_t, hid_t = _make_gru(B, L, E, H)(
        lengths[None, :],
        W_ih,
        W_hh,
        b_ih[:, None],
        b_hh[:, None],
        x,
    )
    outputs = jnp.transpose(out_t, (2, 0, 1))
    hidden = jnp.transpose(hid_t, (1, 0))[None, :, :]
    return outputs, hidden
